# baseline (speedup 1.0000x reference)
import functools

import jax
import jax.numpy as jnp
from jax.experimental import pallas as pl
from jax.experimental.pallas import tpu as pltpu

_TN = 512
_TK = 1024


def _dcn_kernel(a_ref, b_ref, labels_ref, loss_ref, bv_ref, bi_ref, *,
                n_kchunks, tk):
    iidx = pl.program_id(0)
    kidx = pl.program_id(1)

    a = a_ref[...]
    b = b_ref[...]
    bsq = b * b
    ones8 = jnp.ones((8, a.shape[1]), jnp.float32)
    b2row = jax.lax.dot_general(
        ones8, bsq, (((1,), (1,)), ((), ())),
        precision=jax.lax.Precision.HIGHEST,
        preferred_element_type=jnp.float32)[0:1, :]
    a2bf = (2.0 * a).astype(jnp.bfloat16)
    bbf = b.astype(jnp.bfloat16)
    ab2 = jax.lax.dot_general(
        a2bf, bbf, (((1,), (1,)), ((), ())),
        preferred_element_type=jnp.float32)
    c = b2row - ab2
    tmin = jnp.min(c, axis=1, keepdims=True)
    targ = jnp.argmin(c, axis=1, keepdims=True).astype(jnp.int32)
    targ = targ + kidx * tk

    @pl.when(kidx == 0)
    def _first():
        bv_ref[...] = tmin
        bi_ref[...] = targ

    @pl.when(kidx > 0)
    def _fold():
        upd = tmin < bv_ref[...]
        bv_ref[...] = jnp.where(upd, tmin, bv_ref[...])
        bi_ref[...] = jnp.where(upd, targ, bi_ref[...])

    @pl.when(kidx == n_kchunks - 1)
    def _finalize():
        labels_ref[...] = bi_ref[...]
        a2 = jnp.sum(a * a, axis=1, keepdims=True)
        mind = jnp.maximum(a2 + bv_ref[...], 0.0)

        @pl.when(iidx == 0)
        def _init():
            loss_ref[0, 0] = 0.0

        loss_ref[0, 0] += jnp.sum(mind)


@jax.jit
def kernel(embedded, centers):
    n, d = embedded.shape
    k, _ = centers.shape
    n_kchunks = k // _TK
    grid = (n // _TN, n_kchunks)
    labels, loss_sum = pl.pallas_call(
        functools.partial(_dcn_kernel, n_kchunks=n_kchunks, tk=_TK),
        grid=grid,
        in_specs=[
            pl.BlockSpec((_TN, d), lambda i, j: (i, 0)),
            pl.BlockSpec((_TK, d), lambda i, j: (j, 0)),
        ],
        out_specs=[
            pl.BlockSpec((_TN, 1), lambda i, j: (i, 0)),
            pl.BlockSpec(memory_space=pltpu.SMEM),
        ],
        out_shape=[
            jax.ShapeDtypeStruct((n, 1), jnp.int32),
            jax.ShapeDtypeStruct((1, 1), jnp.float32),
        ],
        scratch_shapes=[
            pltpu.VMEM((_TN, 1), jnp.float32),
            pltpu.VMEM((_TN, 1), jnp.int32),
        ],
        compiler_params=pltpu.CompilerParams(
            dimension_semantics=("arbitrary", "arbitrary"),
        ),
    )(embedded, centers)
    loss = loss_sum[0, 0] / n
    return (loss, labels.reshape(n))
